# baseline (speedup 1.0000x reference)
import jax
import jax.numpy as jnp
from jax.experimental import pallas as pl
from jax.experimental.pallas import tpu as pltpu

_B, _C, _H, _W = 8, 384, 200, 176
_HW = _H * _W
_O_PAD = 32
_NW = 3200


def _head_kernel(x_ref, w_ref, b_ref, cls_ref, reg_ref, dir_ref):
    xb = x_ref[0]
    acc = jax.lax.dot_general(
        w_ref[...], xb,
        dimension_numbers=(((1,), (0,)), ((), ())),
        preferred_element_type=jnp.float32,
    )
    acc = acc + b_ref[...]
    cls_ref[0] = acc[0:2]
    reg_ref[0] = acc[2:16]
    dir_ref[0] = acc[16:20]


def kernel(x, W_cls, b_cls, W_reg, b_reg, W_dir, b_dir):
    w = jnp.concatenate([W_cls, W_reg, W_dir], axis=1).T
    w = jnp.pad(w, ((0, _O_PAD - w.shape[0]), (0, 0)))
    b = jnp.concatenate([b_cls, b_reg, b_dir])
    b = jnp.pad(b, (0, _O_PAD - b.shape[0]))[:, None]

    x3 = x.reshape(_B, _C, _HW)
    n_blocks = _HW // _NW

    cls_o, reg_o, dir_o = pl.pallas_call(
        _head_kernel,
        grid=(_B, n_blocks),
        in_specs=[
            pl.BlockSpec((1, _C, _NW), lambda bi, ni: (bi, 0, ni)),
            pl.BlockSpec((_O_PAD, _C), lambda bi, ni: (0, 0)),
            pl.BlockSpec((_O_PAD, 1), lambda bi, ni: (0, 0)),
        ],
        out_specs=[
            pl.BlockSpec((1, 2, _NW), lambda bi, ni: (bi, 0, ni)),
            pl.BlockSpec((1, 14, _NW), lambda bi, ni: (bi, 0, ni)),
            pl.BlockSpec((1, 4, _NW), lambda bi, ni: (bi, 0, ni)),
        ],
        out_shape=[
            jax.ShapeDtypeStruct((_B, 2, _HW), jnp.float32),
            jax.ShapeDtypeStruct((_B, 14, _HW), jnp.float32),
            jax.ShapeDtypeStruct((_B, 4, _HW), jnp.float32),
        ],
        compiler_params=pltpu.CompilerParams(
            dimension_semantics=("parallel", "parallel"),
        ),
    )(x3, w, b)

    cls_score = cls_o.reshape(_B, 2, _H, _W)
    bbox_pred = reg_o.reshape(_B, 14, _H, _W)
    dir_cls_preds = dir_o.reshape(_B, 4, _H, _W)
    return (cls_score, bbox_pred, dir_cls_preds)
